# Initial kernel scaffold; baseline (speedup 1.0000x reference)
import jax
import jax.numpy as jnp
from jax.experimental import pallas as pl


def kernel(feats_A, dist_T, pca_C2, pgat_Wsrc, pgat_Wdst, pgat_al, pgat_ar, pgat_b, bt0_gat_W, bt0_gat_al, bt0_gat_ar, bt0_gat_b, bt0_gin_W1, bt0_gin_b1, bt0_gin_g, bt0_gin_be, bt0_gin_W2, bt0_gin_b2, bt1_gat_W, bt1_gat_al, bt1_gat_ar, bt1_gat_b, bt1_gin_W1, bt1_gin_b1, bt1_gin_g, bt1_gin_be, bt1_gin_W2, bt1_gin_b2, h20_W1, h20_b1, h20_g, h20_be, h20_W2, h20_b2, h21_W1, h21_b1, h21_g, h21_be, h21_W2, h21_b2, out_W1, out_b1, out_W2, out_b2, b0_src, b0_dst, b1_src, b1_dst, g1_dst, i2_src, i2_dst, gid_A, gid_C2):
    raise NotImplementedError("write your pallas kernel here")


# speedup vs baseline: 1.0610x; 1.0610x over previous
import functools

import jax
import jax.numpy as jnp
from jax import lax
from jax.experimental import pallas as pl
from jax.experimental.pallas import tpu as pltpu

N_A = 50000
N_C2 = 5000
B = 64
E_B = 800000
E_I = 80000
IN = 128
HID = 64
HEADS = 4
VF = 5
PCA = 16
BOTIN = VF * HEADS + IN - 3
OH = HID // HEADS


def _bn(x, g, b):
    return g * (x - x.mean(0)) / jnp.sqrt(x.var(0) + 1e-5) + b


def _mlp(x, W1, b1, g, be, W2, b2):
    h = jax.nn.relu(_bn(x @ W1 + b1, g, be))
    return h @ W2 + b2


def _gat_branch(feats, src, dst, gW, gal, gar, gb):
    fs = (feats @ gW).reshape(-1, HEADS, OH)
    el = (fs * gal[None]).sum(-1)
    er = (fs * gar[None]).sum(-1)
    e = jax.nn.leaky_relu(el[src] + er[dst], 0.2)
    ex = jnp.exp(e)
    s = jax.ops.segment_sum(ex, dst, num_segments=N_A)
    a = ex / (s[dst] + 1e-9)
    r = jax.ops.segment_sum(a[..., None] * fs[src], dst, num_segments=N_A)
    r = r + gb.reshape(1, HEADS, OH)
    return jax.nn.relu(r).reshape(N_A, -1)


def _gin(x, src, dst, n, W1, b1, g, be, W2, b2):
    agg = jax.ops.segment_sum(x[src], dst, num_segments=n)
    return jax.nn.relu(_mlp(x + agg, W1, b1, g, be, W2, b2))


def _final_mlp_kernel(hh_ref, W1_ref, b1_ref, W2_ref, b2_ref, out_ref):
    h = jnp.maximum(
        jnp.dot(hh_ref[...], W1_ref[...], preferred_element_type=jnp.float32)
        + b1_ref[...], 0.0)
    out_ref[...] = (
        jnp.dot(h, W2_ref[...], preferred_element_type=jnp.float32) + b2_ref[...])


def _final_mlp(hh, W1, b1, W2, b2):
    return pl.pallas_call(
        _final_mlp_kernel,
        out_shape=jax.ShapeDtypeStruct((B, 1), jnp.float32),
    )(hh, W1, b1.reshape(1, -1), W2, b2.reshape(1, -1))


def kernel(feats_A, dist_T, pca_C2, pgat_Wsrc, pgat_Wdst, pgat_al, pgat_ar, pgat_b, bt0_gat_W, bt0_gat_al, bt0_gat_ar, bt0_gat_b, bt0_gin_W1, bt0_gin_b1, bt0_gin_g, bt0_gin_be, bt0_gin_W2, bt0_gin_b2, bt1_gat_W, bt1_gat_al, bt1_gat_ar, bt1_gat_b, bt1_gin_W1, bt1_gin_b1, bt1_gin_g, bt1_gin_be, bt1_gin_W2, bt1_gin_b2, h20_W1, h20_b1, h20_g, h20_be, h20_W2, h20_b2, h21_W1, h21_b1, h21_g, h21_be, h21_W2, h21_b2, out_W1, out_b1, out_W2, out_b2, b0_src, b0_dst, b1_src, b1_dst, g1_dst, i2_src, i2_dst, gid_A, gid_C2):
    ptab = (dist_T @ pgat_Wsrc).reshape(B, HEADS * VF)
    h = ptab[gid_A] + pgat_b[None, :]
    feats = jnp.concatenate([h, feats_A[:, 3:]], axis=-1)

    hs = []
    for (src, dst, gW, gal, gar, gb, W1, b1, g, be, W2, b2) in (
        (b0_src, b0_dst, bt0_gat_W, bt0_gat_al, bt0_gat_ar, bt0_gat_b,
         bt0_gin_W1, bt0_gin_b1, bt0_gin_g, bt0_gin_be, bt0_gin_W2, bt0_gin_b2),
        (b1_src, b1_dst, bt1_gat_W, bt1_gat_al, bt1_gat_ar, bt1_gat_b,
         bt1_gin_W1, bt1_gin_b1, bt1_gin_g, bt1_gin_be, bt1_gin_W2, bt1_gin_b2),
    ):
        hr = _gat_branch(feats, src, dst, gW, gal, gar, gb)
        hr = _gin(hr, src, dst, N_A, W1, b1, g, be, W2, b2)
        hs.append(hr)
    hA = jnp.concatenate(hs, axis=-1)

    hC = jax.ops.segment_max(hA, g1_dst, num_segments=N_C2)
    hC = jnp.where(jnp.isfinite(hC), hC, 0.0)
    hc = jnp.concatenate([hC, pca_C2], axis=-1)
    hc = _gin(hc, i2_src, i2_dst, N_C2, h20_W1, h20_b1, h20_g, h20_be, h20_W2, h20_b2)
    hc = _gin(hc, i2_src, i2_dst, N_C2, h21_W1, h21_b1, h21_g, h21_be, h21_W2, h21_b2)

    r1 = jax.ops.segment_sum(hA, gid_A, num_segments=B)
    r2 = jax.ops.segment_sum(hc, gid_C2, num_segments=B)
    hh = jnp.concatenate([r1, r2], axis=-1)
    return _final_mlp(hh, out_W1, out_b1, out_W2, out_b2)


# speedup vs baseline: 8.8597x; 8.3507x over previous
import functools

import jax
import jax.numpy as jnp
from jax import lax
from jax.experimental import pallas as pl
from jax.experimental.pallas import tpu as pltpu
from jax.experimental.pallas import tpu_sc as plsc

N_A = 50000
N_C2 = 5000
B = 64
E_B = 800000
HID = 64
HEADS = 4
VF = 5
PCA = 16
OH = HID // HEADS

NC = 2
NS = 16
NW = NC * NS

RB_A = 2000
RB_E = 10000

SPMEM_BUDGET_WORDS = 1_400_000


def _ceil_to(x, m):
    return (x + m - 1) // m * m


@functools.cache
def _make_gather_rows(d, e_pad, cho):
    k = cho // 128
    n_chunks = e_pad // cho
    mesh = plsc.VectorSubcoreMesh(core_axis_name="c", subcore_axis_name="s")

    @functools.partial(
        pl.kernel, mesh=mesh,
        out_type=jax.ShapeDtypeStruct((e_pad, d), jnp.float32),
        compiler_params=pltpu.CompilerParams(use_tc_tiling_on_sc=False),
        scratch_types=[
            pltpu.VMEM((k, 128), jnp.int32),
            pltpu.VMEM((cho, d), jnp.float32),
            pltpu.SemaphoreType.DMA,
        ],
    )
    def kern(tab_hbm, idx_hbm, out_hbm, idx_v, rows_v, sem):
        w = lax.axis_index("s") * NC + lax.axis_index("c")
        n_mine = (n_chunks - 1 - w) // NW + 1

        def body(t, carry):
            c0 = w + t * NW
            pltpu.sync_copy(idx_hbm.at[pl.ds(c0 * k, k)], idx_v)
            descs = [
                pltpu.async_copy(tab_hbm.at[idx_v.at[j]],
                                 rows_v.at[pl.ds(j * 128, 128)], sem)
                for j in range(k)
            ]
            for dsc in descs:
                dsc.wait()
            pltpu.sync_copy(rows_v, out_hbm.at[pl.ds(c0 * cho, cho)])
            return carry

        lax.fori_loop(0, n_mine, body, 0)

    return kern


@functools.cache
def _make_scatter_add(n_pad, d, e_pad, cho):
    k = cho // 128
    n_chunks = e_pad // cho
    rpw = n_pad // NS
    mesh = plsc.VectorSubcoreMesh(core_axis_name="c", subcore_axis_name="s")

    @functools.partial(
        pl.kernel, mesh=mesh,
        out_type=jax.ShapeDtypeStruct((NC, n_pad, d), jnp.float32),
        compiler_params=pltpu.CompilerParams(use_tc_tiling_on_sc=False),
        scratch_types=[
            pltpu.VMEM((k, 128), jnp.int32),
            pltpu.VMEM((cho, d), jnp.float32),
            pltpu.VMEM_SHARED((n_pad, d), jnp.float32),
            pltpu.SemaphoreType.DMA,
        ],
    )
    def kern(rows_hbm, idx_hbm, zeros_hbm, out_hbm, idx_v, rows_v, acc_sh, sem):
        cid = lax.axis_index("c")
        sid = lax.axis_index("s")
        w = sid * NC + cid
        pltpu.sync_copy(zeros_hbm.at[pl.ds(sid * rpw, rpw)],
                        acc_sh.at[pl.ds(sid * rpw, rpw)])
        plsc.subcore_barrier()
        n_mine = (n_chunks - 1 - w) // NW + 1

        def body(t, carry):
            c0 = w + t * NW
            pltpu.sync_copy(idx_hbm.at[pl.ds(c0 * k, k)], idx_v)
            pltpu.sync_copy(rows_hbm.at[pl.ds(c0 * cho, cho)], rows_v)
            descs = [
                pltpu.async_copy(rows_v.at[pl.ds(j * 128, 128)],
                                 acc_sh.at[idx_v.at[j]], sem, add=True)
                for j in range(k)
            ]
            for dsc in descs:
                dsc.wait()
            return carry

        lax.fori_loop(0, n_mine, body, 0)
        plsc.subcore_barrier()
        pltpu.sync_copy(acc_sh.at[pl.ds(sid * rpw, rpw)],
                        out_hbm.at[cid, pl.ds(sid * rpw, rpw)])

    return kern


def _pad_idx(idx, e_pad):
    idx = idx.astype(jnp.int32)
    return jnp.pad(idx, (0, e_pad - idx.shape[0])).reshape(-1, 128)


def _sc_gather(tab, idx):
    e, d = idx.shape[0], tab.shape[1]
    if d < 8:
        tab = jnp.pad(tab, ((0, 0), (0, 8 - d)))
        return _sc_gather(tab, idx)[:, :d]
    cho = 512 if d > 64 else 1024
    e_pad = _ceil_to(e, NW * cho)
    out = _make_gather_rows(d, e_pad, cho)(tab, _pad_idx(idx, e_pad))
    return out[:e]


def _sc_scatter_parts(rows, idx, n):
    e, d = rows.shape
    if d < 8:
        rows = jnp.pad(rows, ((0, 0), (0, 8 - d)))
        d = 8
    n_pad = _ceil_to(n, NS)
    dc = d if n_pad * d <= SPMEM_BUDGET_WORDS else 16
    cho = 512 if d > 64 else 1024
    e_pad = _ceil_to(e, NW * cho)
    idx_p = _pad_idx(idx, e_pad)
    parts = []
    for c in range(0, d, dc):
        rp = jnp.pad(rows[:, c:c + dc], ((0, e_pad - e), (0, 0)))
        zeros = jnp.zeros((n_pad, rp.shape[1]), jnp.float32)
        parts.append(_make_scatter_add(n_pad, rp.shape[1], e_pad, cho)(
            rp, idx_p, zeros))
    return parts


def _full(shape):
    return pl.BlockSpec(shape, lambda *_: tuple(0 for _ in shape))


def _rows(shape, axis=0):
    def imap(i):
        return tuple(i if a == axis else 0 for a in range(len(shape)))
    return pl.BlockSpec(shape, imap)


def _pre_kernel(gidf_ref, fA_ref, distT_ref, pWsrc_ref, pb_ref,
                gW0_ref, gW1_ref, al0_ref, ar0_ref, al1_ref, ar1_ref,
                fs0_ref, fs1_ref, el0_ref, er0_ref, el1_ref, er1_ref):
    ptab = jnp.dot(distT_ref[...], pWsrc_ref[...],
                   preferred_element_type=jnp.float32)
    iota = lax.broadcasted_iota(jnp.int32, (RB_A, B), 1).astype(jnp.float32)
    onehot = (gidf_ref[...] == iota).astype(jnp.float32)
    h = jnp.dot(onehot, ptab, preferred_element_type=jnp.float32) + pb_ref[...]
    feats = jnp.concatenate([h, fA_ref[...][:, 3:]], axis=-1)
    for gW_ref, al_ref, ar_ref, fs_ref, el_ref, er_ref in (
        (gW0_ref, al0_ref, ar0_ref, fs0_ref, el0_ref, er0_ref),
        (gW1_ref, al1_ref, ar1_ref, fs1_ref, el1_ref, er1_ref),
    ):
        fs = jnp.dot(feats, gW_ref[...], preferred_element_type=jnp.float32)
        fs_ref[...] = fs
        el_ref[...] = jnp.dot(fs, al_ref[...], preferred_element_type=jnp.float32)
        er_ref[...] = jnp.dot(fs, ar_ref[...], preferred_element_type=jnp.float32)


def _pre(gid_A, feats_A, dist_T, pgat_Wsrc, pgat_b, gW0, gW1, Al0, Ar0, Al1, Ar1):
    gidf = gid_A.astype(jnp.float32)[:, None]
    outs = [jax.ShapeDtypeStruct((N_A, HID), jnp.float32),
            jax.ShapeDtypeStruct((N_A, HID), jnp.float32)] + \
           [jax.ShapeDtypeStruct((N_A, 8), jnp.float32)] * 4
    return pl.pallas_call(
        _pre_kernel,
        grid=(N_A // RB_A,),
        in_specs=[_rows((RB_A, 1)), _rows((RB_A, 128)), _full((B, VF)),
                  _full((VF, HEADS * VF)), _full((1, HEADS * VF)),
                  _full((145, HID)), _full((145, HID)),
                  _full((HID, 8)), _full((HID, 8)), _full((HID, 8)),
                  _full((HID, 8))],
        out_specs=[_rows((RB_A, HID)), _rows((RB_A, HID))] + [_rows((RB_A, 8))] * 4,
        out_shape=outs,
    )(gidf, feats_A, dist_T, pgat_Wsrc, pgat_b.reshape(1, -1),
      gW0, gW1, Al0, Ar0, Al1, Ar1)


def _ex_kernel(el_ref, er_ref, ex_ref):
    e = el_ref[...] + er_ref[...]
    e = jnp.maximum(e, 0.2 * e)
    iota = lax.broadcasted_iota(jnp.int32, (RB_E, 8), 1)
    ex_ref[...] = jnp.where(iota < HEADS, jnp.exp(e), 0.0)


def _ex(el, er, e_n):
    return pl.pallas_call(
        _ex_kernel,
        grid=(e_n // RB_E,),
        in_specs=[_rows((RB_E, 8))] * 2,
        out_specs=_rows((RB_E, 8)),
        out_shape=jax.ShapeDtypeStruct((e_n, 8), jnp.float32),
    )(el, er)


def _msg_kernel(ex_ref, fs_ref, S_ref, msg_ref):
    w = jnp.dot(ex_ref[...], S_ref[...], preferred_element_type=jnp.float32)
    msg_ref[...] = w * fs_ref[...]


def _msg(ex, fs_src, S, e_n):
    return pl.pallas_call(
        _msg_kernel,
        grid=(e_n // RB_E,),
        in_specs=[_rows((RB_E, 8)), _rows((RB_E, HID)), _full((8, HID))],
        out_specs=_rows((RB_E, HID)),
        out_shape=jax.ShapeDtypeStruct((e_n, HID), jnp.float32),
    )(ex, fs_src, S)


def _head_expand():
    return jnp.concatenate(
        [jnp.where(lax.broadcasted_iota(jnp.int32, (HEADS, OH), 0) == h,
                   1.0, 0.0) for h in range(HEADS)], axis=-1)


def _gatfin_kernel(s_ref, u0_ref, u1_ref, u2_ref, u3_ref, gb_ref, x_ref):
    s = s_ref[...][0] + s_ref[...][1]
    u = jnp.concatenate(
        [u_ref[...][0] + u_ref[...][1]
         for u_ref in (u0_ref, u1_ref, u2_ref, u3_ref)], axis=-1)
    sden = 1.0 / (s[:, :HEADS] + 1e-9)
    w = jnp.dot(sden, _head_expand(), preferred_element_type=jnp.float32)
    x_ref[...] = jnp.maximum(u * w + gb_ref[...], 0.0)


def _gatfin(s_parts, u_parts, gb):
    return pl.pallas_call(
        _gatfin_kernel,
        grid=(N_A // RB_A,),
        in_specs=[_rows((NC, RB_A, 8), axis=1)] +
                 [_rows((NC, RB_A, 16), axis=1)] * 4 + [_full((1, HID))],
        out_specs=_rows((RB_A, HID)),
        out_shape=jax.ShapeDtypeStruct((N_A, HID), jnp.float32),
    )(s_parts, *u_parts, gb.reshape(1, -1))


@functools.cache
def _make_gin1(n, d_in, rb, nparts, dc):
    grid = n // rb

    def kern(*refs):
        x_ref = refs[0]
        part_refs = refs[1:1 + nparts]
        W1_ref, b1_ref = refs[1 + nparts], refs[2 + nparts]
        y_ref, s_ref, s2_ref = refs[3 + nparts:]
        agg = jnp.concatenate(
            [p[...][0] + p[...][1] for p in part_refs], axis=-1)
        y = jnp.dot(x_ref[...] + agg[:, :d_in], W1_ref[...],
                    preferred_element_type=jnp.float32) + b1_ref[...]
        y_ref[...] = y
        ps = jnp.sum(y, axis=0, keepdims=True)
        ps2 = jnp.sum(y * y, axis=0, keepdims=True)
        if grid > 1:
            @pl.when(pl.program_id(0) == 0)
            def _init():
                s_ref[...] = ps
                s2_ref[...] = ps2

            @pl.when(pl.program_id(0) != 0)
            def _acc():
                s_ref[...] += ps
                s2_ref[...] += ps2
        else:
            s_ref[...] = ps
            s2_ref[...] = ps2

    return pl.pallas_call(
        kern,
        grid=(grid,),
        in_specs=[_rows((rb, d_in))] +
                 [_rows((NC, rb, dc), axis=1)] * nparts +
                 [_full((d_in, HID)), _full((1, HID))],
        out_specs=[_rows((rb, HID)), _full((1, HID)), _full((1, HID))],
        out_shape=[jax.ShapeDtypeStruct((n, HID), jnp.float32),
                   jax.ShapeDtypeStruct((1, HID), jnp.float32),
                   jax.ShapeDtypeStruct((1, HID), jnp.float32)],
    )


@functools.cache
def _make_gin2(n, rb):
    grid = n // rb
    inv_n = 1.0 / n

    def kern(y_ref, s_ref, s2_ref, g_ref, be_ref, W2_ref, b2_ref, out_ref):
        mean = s_ref[...] * inv_n
        var = s2_ref[...] * inv_n - mean * mean
        scale = g_ref[...] * lax.rsqrt(var + 1e-5)
        h = jnp.maximum((y_ref[...] - mean) * scale + be_ref[...], 0.0)
        out = jnp.dot(h, W2_ref[...], preferred_element_type=jnp.float32)
        out_ref[...] = jnp.maximum(out + b2_ref[...], 0.0)

    return pl.pallas_call(
        kern,
        grid=(grid,),
        in_specs=[_rows((rb, HID)), _full((1, HID)), _full((1, HID)),
                  _full((1, HID)), _full((1, HID)), _full((HID, HID)),
                  _full((1, HID))],
        out_specs=_rows((rb, HID)),
        out_shape=jax.ShapeDtypeStruct((n, HID), jnp.float32),
    )


def _gin_mlp(x, agg_parts, W1, b1, g, be, W2, b2, n, rb):
    d_in = x.shape[1]
    dc = agg_parts[0].shape[2]
    y, s, s2 = _make_gin1(n, d_in, rb, len(agg_parts), dc)(
        x, *agg_parts, W1, b1.reshape(1, -1))
    return _make_gin2(n, rb)(y, s, s2, g.reshape(1, -1), be.reshape(1, -1),
                             W2, b2.reshape(1, -1))


@functools.cache
def _make_read(n, d, rb):
    grid = n // rb

    def kern(gidf_ref, x_ref, out_ref):
        iota = lax.broadcasted_iota(jnp.int32, (rb, B), 1).astype(jnp.float32)
        onehot = (gidf_ref[...] == iota).astype(jnp.float32)
        r = jnp.dot(onehot.T, x_ref[...], preferred_element_type=jnp.float32)
        if grid > 1:
            @pl.when(pl.program_id(0) == 0)
            def _init():
                out_ref[...] = r

            @pl.when(pl.program_id(0) != 0)
            def _acc():
                out_ref[...] += r
        else:
            out_ref[...] = r

    return pl.pallas_call(
        kern,
        grid=(grid,),
        in_specs=[_rows((rb, 1)), _rows((rb, d))],
        out_specs=_full((B, d)),
        out_shape=jax.ShapeDtypeStruct((B, d), jnp.float32),
    )


def _final_kernel(r1_ref, r2_ref, W1a_ref, W1b_ref, b1_ref, W2_ref, b2_ref,
                  out_ref):
    h = (jnp.dot(r1_ref[...], W1a_ref[...], preferred_element_type=jnp.float32)
         + jnp.dot(r2_ref[...], W1b_ref[...], preferred_element_type=jnp.float32)
         + b1_ref[...])
    h = jnp.maximum(h, 0.0)
    out_ref[...] = jnp.dot(h, W2_ref[...],
                           preferred_element_type=jnp.float32) + b2_ref[...]


def _final(r1, r2, out_W1, out_b1, out_W2, out_b2):
    return pl.pallas_call(
        _final_kernel,
        in_specs=[_full((B, 2 * HID)), _full((B, HID)), _full((2 * HID, HID)),
                  _full((HID, HID)), _full((1, HID)), _full((HID, 1)),
                  _full((1, 1))],
        out_specs=_full((B, 1)),
        out_shape=jax.ShapeDtypeStruct((B, 1), jnp.float32),
    )(r1, r2, out_W1[:2 * HID], out_W1[2 * HID:], out_b1.reshape(1, -1),
      out_W2, out_b2.reshape(1, -1))


def _expand_attn(a):
    m = jnp.zeros((HID, 8), jnp.float32)
    for h in range(HEADS):
        m = m.at[h * OH:(h + 1) * OH, h].set(a[h])
    return m


def kernel(feats_A, dist_T, pca_C2, pgat_Wsrc, pgat_Wdst, pgat_al, pgat_ar, pgat_b, bt0_gat_W, bt0_gat_al, bt0_gat_ar, bt0_gat_b, bt0_gin_W1, bt0_gin_b1, bt0_gin_g, bt0_gin_be, bt0_gin_W2, bt0_gin_b2, bt1_gat_W, bt1_gat_al, bt1_gat_ar, bt1_gat_b, bt1_gin_W1, bt1_gin_b1, bt1_gin_g, bt1_gin_be, bt1_gin_W2, bt1_gin_b2, h20_W1, h20_b1, h20_g, h20_be, h20_W2, h20_b2, h21_W1, h21_b1, h21_g, h21_be, h21_W2, h21_b2, out_W1, out_b1, out_W2, out_b2, b0_src, b0_dst, b1_src, b1_dst, g1_dst, i2_src, i2_dst, gid_A, gid_C2):
    fs0, fs1, el0, er0, el1, er1 = _pre(
        gid_A, feats_A, dist_T, pgat_Wsrc, pgat_b, bt0_gat_W, bt1_gat_W,
        _expand_attn(bt0_gat_al), _expand_attn(bt0_gat_ar),
        _expand_attn(bt1_gat_al), _expand_attn(bt1_gat_ar))

    S = jnp.pad(_head_expand(), ((0, 8 - HEADS), (0, 0)))

    hs = []
    for (src, dst, fs, eltab, ertab, gb, W1, b1, g, be, W2, b2) in (
        (b0_src, b0_dst, fs0, el0, er0, bt0_gat_b,
         bt0_gin_W1, bt0_gin_b1, bt0_gin_g, bt0_gin_be, bt0_gin_W2, bt0_gin_b2),
        (b1_src, b1_dst, fs1, el1, er1, bt1_gat_b,
         bt1_gin_W1, bt1_gin_b1, bt1_gin_g, bt1_gin_be, bt1_gin_W2, bt1_gin_b2),
    ):
        el = _sc_gather(eltab, src)
        er = _sc_gather(ertab, dst)
        ex = _ex(el, er, E_B)
        s_parts = _sc_scatter_parts(ex, dst, N_A)[0]
        fs_src = _sc_gather(fs, src)
        msg = _msg(ex, fs_src, S, E_B)
        u_parts = _sc_scatter_parts(msg, dst, N_A)
        x = _gatfin(s_parts, u_parts, gb)
        x_src = _sc_gather(x, src)
        agg_parts = _sc_scatter_parts(x_src, dst, N_A)
        hr = _gin_mlp(x, agg_parts, W1, b1, g, be, W2, b2, N_A, RB_A)
        hs.append(hr)
    hA = jnp.concatenate(hs, axis=-1)

    hC = jax.ops.segment_max(hA, g1_dst, num_segments=N_C2)
    hC = jnp.where(jnp.isfinite(hC), hC, 0.0)
    hc = jnp.concatenate([hC, pca_C2], axis=-1)

    for (W1, b1, g, be, W2, b2) in (
        (h20_W1, h20_b1, h20_g, h20_be, h20_W2, h20_b2),
        (h21_W1, h21_b1, h21_g, h21_be, h21_W2, h21_b2),
    ):
        x_src = _sc_gather(hc, i2_src)
        agg_parts = _sc_scatter_parts(x_src, i2_dst, N_C2)
        agg_parts = [p[:, :N_C2, :] for p in agg_parts]
        hc = _gin_mlp(hc, agg_parts, W1, b1, g, be, W2, b2, N_C2, N_C2)

    gidf_A = gid_A.astype(jnp.float32)[:, None]
    gidf_C = gid_C2.astype(jnp.float32)[:, None]
    r1 = _make_read(N_A, 2 * HID, RB_A)(gidf_A, hA)
    r2 = _make_read(N_C2, HID, N_C2)(gidf_C, hc)
    return _final(r1, r2, out_W1, out_b1, out_W2, out_b2)
